# baseline (speedup 1.0000x reference)
import functools

import jax
import jax.numpy as jnp
from jax import lax
from jax.experimental import pallas as pl
from jax.experimental.pallas import tpu as pltpu
from jax.experimental.pallas import tpu_sc as plsc

B, T, V = 1024, 50, 1000
N = B * T
NC, NS, L = 2, 16, 16
NW = NC * NS
R = N // NW
CH = R // L


def _nll_body(inp_hbm, tgt_hbm, msk_hbm, loss_out, mask_out,
              tgt_v, msk_v, idx_v, gat_v, stage_v, sem):
    cid = lax.axis_index("c")
    sid = lax.axis_index("s")
    wid = cid * NS + sid
    base = wid * R

    pltpu.sync_copy(tgt_hbm.at[pl.ds(base, R)], tgt_v)
    pltpu.sync_copy(msk_hbm.at[pl.ds(base, R)], msk_v)

    lane = lax.iota(jnp.int32, L)

    def idx_body(j, _):
        off = j * L
        t16 = jnp.maximum(tgt_v[pl.ds(off, L)] - 1, 0)
        idx_v[pl.ds(off, L)] = (base + off + lane) * V + t16
        return 0

    lax.fori_loop(0, CH, idx_body, 0)

    pltpu.async_copy(inp_hbm.at[idx_v], gat_v, sem).wait()

    def red_body(j, carry):
        acc_l, acc_m = carry
        off = j * L
        m = msk_v[pl.ds(off, L)]
        return acc_l + gat_v[pl.ds(off, L)] * m, acc_m + m

    zero = jnp.zeros((L,), jnp.float32)
    acc_l, acc_m = lax.fori_loop(0, CH, red_body, (zero, zero))

    stage_v[...] = acc_l
    pltpu.sync_copy(stage_v, loss_out.at[wid])
    stage_v[...] = acc_m
    pltpu.sync_copy(stage_v, mask_out.at[wid])


_nll_kernel = functools.partial(
    pl.kernel,
    out_type=[
        jax.ShapeDtypeStruct((NW, L), jnp.float32),
        jax.ShapeDtypeStruct((NW, L), jnp.float32),
    ],
    mesh=plsc.VectorSubcoreMesh(core_axis_name="c", subcore_axis_name="s"),
    scratch_types=[
        pltpu.VMEM((R,), jnp.int32),
        pltpu.VMEM((R,), jnp.float32),
        pltpu.VMEM((R,), jnp.int32),
        pltpu.VMEM((R,), jnp.float32),
        pltpu.VMEM((L,), jnp.float32),
        pltpu.SemaphoreType.DMA,
    ],
)(_nll_body)


@jax.jit
def kernel(input, target, mask):
    Tt = input.shape[1]
    tgt = target[:, :Tt].reshape(-1).astype(jnp.int32)
    msk = mask[:, :Tt].reshape(-1).astype(jnp.float32)
    loss_p, mask_p = _nll_kernel(input.reshape(-1), tgt, msk)
    return -jnp.sum(loss_p) / jnp.sum(mask_p)


# speedup vs baseline: 1.5928x; 1.5928x over previous
import functools

import jax
import jax.numpy as jnp
from jax import lax
from jax.experimental import pallas as pl
from jax.experimental.pallas import tpu as pltpu

B, T, V = 1024, 50, 1000
BB = 16
GRID = B // BB


def _nll_body(inp_ref, tgt_ref, msk_ref, out_ref, acc_ref):
    pid = pl.program_id(0)

    t = jnp.maximum(tgt_ref[...] - 1, 0)
    m = msk_ref[...]
    x = inp_ref[...]
    vio = lax.broadcasted_iota(jnp.int32, (BB, T, V), 2)
    sel = jnp.where(vio == t[:, :, None], x, 0.0)
    g = jnp.sum(sel, axis=2)
    part_l = jnp.sum(g * m)
    part_m = jnp.sum(m)

    @pl.when(pid == 0)
    def _():
        acc_ref[0] = 0.0
        acc_ref[1] = 0.0

    acc_ref[0] = acc_ref[0] + part_l
    acc_ref[1] = acc_ref[1] + part_m

    @pl.when(pid == GRID - 1)
    def _():
        out_ref[...] = jnp.full((8, 128), -acc_ref[0] / acc_ref[1],
                                jnp.float32)


_nll_call = pl.pallas_call(
    _nll_body,
    grid=(GRID,),
    in_specs=[
        pl.BlockSpec((BB, T, V), lambda i: (i, 0, 0)),
        pl.BlockSpec((BB, T), lambda i: (i, 0)),
        pl.BlockSpec((BB, T), lambda i: (i, 0)),
    ],
    out_specs=pl.BlockSpec((8, 128), lambda i: (0, 0)),
    out_shape=jax.ShapeDtypeStruct((8, 128), jnp.float32),
    scratch_shapes=[pltpu.SMEM((2,), jnp.float32)],
)


@jax.jit
def kernel(input, target, mask):
    Tt = input.shape[1]
    tgt = target[:, :Tt].astype(jnp.int32)
    msk = mask[:, :Tt].astype(jnp.float32)
    return _nll_call(input, tgt, msk)[0, 0]


# speedup vs baseline: 1.7347x; 1.0891x over previous
import functools

import jax
import jax.numpy as jnp
from jax import lax
from jax.experimental import pallas as pl
from jax.experimental.pallas import tpu as pltpu

B, T, V = 1024, 50, 1000
BB = 64
GRID = B // BB


def _nll_body(inp_ref, tgt_ref, msk_ref, out_ref, acc_ref):
    pid = pl.program_id(0)

    t = jnp.maximum(tgt_ref[...] - 1, 0)
    m = msk_ref[...]
    x = inp_ref[...]
    vio = lax.broadcasted_iota(jnp.int32, (BB, T, V), 2)
    sel = jnp.where(vio == t[:, :, None], x, 0.0)
    g = jnp.sum(sel, axis=2)
    part_l = jnp.sum(g * m)
    part_m = jnp.sum(m)

    @pl.when(pid == 0)
    def _():
        acc_ref[0] = 0.0
        acc_ref[1] = 0.0

    acc_ref[0] = acc_ref[0] + part_l
    acc_ref[1] = acc_ref[1] + part_m

    @pl.when(pid == GRID - 1)
    def _():
        out_ref[...] = jnp.full((8, 128), -acc_ref[0] / acc_ref[1],
                                jnp.float32)


_nll_call = pl.pallas_call(
    _nll_body,
    grid=(GRID,),
    in_specs=[
        pl.BlockSpec((BB, T, V), lambda i: (i, 0, 0)),
        pl.BlockSpec((BB, T), lambda i: (i, 0)),
        pl.BlockSpec((BB, T), lambda i: (i, 0)),
    ],
    out_specs=pl.BlockSpec((8, 128), lambda i: (0, 0)),
    out_shape=jax.ShapeDtypeStruct((8, 128), jnp.float32),
    scratch_shapes=[pltpu.SMEM((2,), jnp.float32)],
)


@jax.jit
def kernel(input, target, mask):
    Tt = input.shape[1]
    tgt = target[:, :Tt].astype(jnp.int32)
    msk = mask[:, :Tt].astype(jnp.float32)
    return _nll_call(input, tgt, msk)[0, 0]
